# baseline (speedup 1.0000x reference)
import jax
import jax.numpy as jnp
from jax import lax
from jax.experimental import pallas as pl
from jax.experimental.pallas import tpu as pltpu
from jax.experimental.pallas import tpu_sc as plsc

_NC, _NS, _L = 2, 16, 16


def _sc_body(col_hbm, row_hbm, out_hbm, colbuf, rowbuf, chunk, sem):
    b, hw, d2 = out_hbm.shape
    d = d2 // 2
    w = col_hbm.shape[0]
    wid = lax.axis_index("s") * _NC + lax.axis_index("c")
    pltpu.sync_copy(col_hbm, colbuf)
    pltpu.sync_copy(row_hbm.at[wid], rowbuf)
    rv = [rowbuf[pl.ds(k * _L, _L)] for k in range(d // _L)]
    for c in range(w):
        for k in range(d // _L):
            chunk[c, pl.ds(k * _L, _L)] = colbuf[c, pl.ds(k * _L, _L)]
            chunk[c, pl.ds(d + k * _L, _L)] = rv[k]
    copies = [
        pltpu.async_copy(chunk, out_hbm.at[i, pl.ds(wid * w, w), :], sem)
        for i in range(b)
    ]
    for cp in copies:
        cp.wait()


def kernel(x, row_embed, col_embed):
    b = x.shape[0]
    h, w = x.shape[-3], x.shape[-2]
    d = row_embed.shape[1]
    assert h == _NC * _NS and w == h and d % _L == 0
    col = col_embed[:w]
    row = row_embed[:h]
    mesh = plsc.VectorSubcoreMesh(core_axis_name="c", subcore_axis_name="s")
    sc = pl.kernel(
        _sc_body,
        out_type=jax.ShapeDtypeStruct((b, h * w, 2 * d), jnp.float32),
        mesh=mesh,
        scratch_types=[
            pltpu.VMEM((w, d), jnp.float32),
            pltpu.VMEM((d,), jnp.float32),
            pltpu.VMEM((w, 2 * d), jnp.float32),
            pltpu.SemaphoreType.DMA,
        ],
    )
    return sc(col, row)
